# baseline (speedup 1.0000x reference)
import functools

import jax
import jax.numpy as jnp
from jax import lax
from jax.experimental import pallas as pl
from jax.experimental.pallas import tpu as pltpu
from jax.experimental.pallas import tpu_sc as plsc

N = 819200
EMB = 64
HALF = EMB // 2
N2 = N // 2

NC = 2
NS = 16
NW = NC * NS
ROWS_PER_W = N // NW
CHUNK = 128
CHUNKS_PER_W = ROWS_PER_W // CHUNK
KBUF = 8
OUTER = CHUNKS_PER_W // KBUF

CP = 32768
GP = N // CP
C2 = 8192
G1 = N2 // C2


def _sc_gather(idx3, table):
    mesh = plsc.VectorSubcoreMesh(core_axis_name="c", subcore_axis_name="s")

    @functools.partial(
        pl.kernel,
        out_type=jax.ShapeDtypeStruct((N, 128), jnp.float32),
        mesh=mesh,
        scratch_types=[
            pltpu.VMEM((CHUNKS_PER_W, CHUNK), jnp.int32),
            pltpu.VMEM((4, CHUNK, 128), jnp.float32),
            pltpu.SemaphoreType.DMA,
            pltpu.SemaphoreType.DMA,
        ],
    )
    def gather_kernel(idx_hbm, table_hbm, out_hbm, idx_v, rows_v, semg, semo):
        wid = lax.axis_index("s") * NC + lax.axis_index("c")
        pltpu.sync_copy(idx_hbm.at[wid], idx_v)
        row0 = wid * ROWS_PER_W

        def body(t, carry):
            @pl.when(t > 0)
            def _():
                for i in range(4):
                    pltpu.make_async_copy(
                        rows_v.at[i],
                        out_hbm.at[pl.ds(0, CHUNK)],
                        semo,
                    ).wait()

            for i in range(4):
                c = t * 4 + i
                pltpu.make_async_copy(
                    table_hbm.at[idx_v.at[c]], rows_v.at[i], semg
                ).start()
            for i in range(4):
                pltpu.make_async_copy(
                    table_hbm.at[idx_v.at[0]], rows_v.at[i], semg
                ).wait()
            for i in range(4):
                c = t * 4 + i
                pltpu.make_async_copy(
                    rows_v.at[i],
                    out_hbm.at[pl.ds(row0 + c * CHUNK, CHUNK)],
                    semo,
                ).start()
            return carry

        lax.fori_loop(0, CHUNKS_PER_W // 4, body, 0)
        for i in range(4):
            pltpu.make_async_copy(
                rows_v.at[i],
                out_hbm.at[pl.ds(0, CHUNK)],
                semo,
            ).wait()

    return gather_kernel(idx3, table)


def _pos_body(idx_ref, fcol_ref, pos_ref):
    x = idx_ref[...].reshape(1, CP).astype(jnp.float32)
    arg = fcol_ref[0:HALF, :] * x
    s = jnp.sin(arg)
    c = jnp.cos(arg)
    pos_ref[...] = jnp.concatenate([s, c], axis=0)


def _half_sel(emb_ref, idx_ref):
    et = emb_ref[...].T
    par = (idx_ref[...] & 1).reshape(1, C2)
    return jnp.where(par == 0, et[0:64, :], et[64:128, :])


def _sums_body(pos_a_ref, pos_b_ref, emb_a_ref, emb_b_ref, idx_a_ref,
               idx_b_ref, out_a_ref, out_b_ref, sum_ref):
    step = pl.program_id(0)
    pos = jnp.concatenate([pos_a_ref[...], pos_b_ref[...]], axis=0)
    emb = jnp.concatenate(
        [_half_sel(emb_a_ref, idx_a_ref), _half_sel(emb_b_ref, idx_b_ref)],
        axis=0,
    )
    out = emb + pos
    out_a_ref[...] = out[0:64, :]
    out_b_ref[...] = out[64:128, :]
    ps = jnp.sum(out, axis=1, keepdims=True)
    pq = jnp.sum(out * out, axis=1, keepdims=True)

    @pl.when(step == 0)
    def _():
        sum_ref[...] = jnp.zeros((128, 2), jnp.float32)

    sum_ref[:, 0:1] += ps
    sum_ref[:, 1:2] += pq


def _norm_body(out_ref, sum_ref, wb_ref, normed_ref):
    s1 = jnp.sum(sum_ref[:, 0:1])
    s2 = jnp.sum(sum_ref[:, 1:2])
    total = float(N * EMB)
    mean = s1 / total
    var = s2 / total - mean * mean
    denom = jnp.sqrt(var) + 1e-5
    scale = wb_ref[0:64, 0:1] / denom
    shift = wb_ref[0:64, 1:2]
    normed_ref[...] = (out_ref[...] - mean) * scale + shift


def _norm_body_b(carry_ref, out_ref, sum_ref, wb_ref, normed_ref):
    del carry_ref
    _norm_body(out_ref, sum_ref, wb_ref, normed_ref)


def kernel(input_tensor, table, weight, bias):
    freq = (1e-4) ** jnp.linspace(0.0, 1.0, HALF)
    freqf = jnp.concatenate([freq, freq])
    fcol = freqf.reshape(EMB, 1).astype(jnp.float32)

    pos_t = pl.pallas_call(
        _pos_body,
        grid=(GP,),
        in_specs=[
            pl.BlockSpec((CP,), lambda i: (i,)),
            pl.BlockSpec((EMB, 1), lambda i: (0, 0)),
        ],
        out_specs=pl.BlockSpec((EMB, CP), lambda i: (0, i)),
        out_shape=jax.ShapeDtypeStruct((EMB, N), jnp.float32),
        compiler_params=pltpu.CompilerParams(
            dimension_semantics=("arbitrary",),
        ),
    )(input_tensor, fcol)

    table2 = table.reshape(-1, 128)
    idx3 = (input_tensor // 2).reshape(NW, CHUNKS_PER_W, CHUNK)
    emb_p = _sc_gather(idx3, table2)

    out_a, out_b, sums = pl.pallas_call(
        _sums_body,
        grid=(G1,),
        in_specs=[
            pl.BlockSpec((EMB, C2), lambda i: (0, i)),
            pl.BlockSpec((EMB, C2), lambda i: (0, G1 + i)),
            pl.BlockSpec((C2, 128), lambda i: (i, 0)),
            pl.BlockSpec((C2, 128), lambda i: (G1 + i, 0)),
            pl.BlockSpec((C2,), lambda i: (i,)),
            pl.BlockSpec((C2,), lambda i: (G1 + i,)),
        ],
        out_specs=[
            pl.BlockSpec((EMB, C2), lambda i: (0, i)),
            pl.BlockSpec((EMB, C2), lambda i: (0, i)),
            pl.BlockSpec((128, 2), lambda i: (0, 0)),
        ],
        out_shape=[
            jax.ShapeDtypeStruct((EMB, N2), jnp.float32),
            jax.ShapeDtypeStruct((EMB, N2), jnp.float32),
            jax.ShapeDtypeStruct((128, 2), jnp.float32),
        ],
        compiler_params=pltpu.CompilerParams(
            dimension_semantics=("arbitrary",),
        ),
    )(pos_t, pos_t, emb_p, emb_p, input_tensor, input_tensor)

    wcol = jnp.concatenate([weight, weight]).reshape(128, 1)
    bcol = jnp.concatenate([bias, bias]).reshape(128, 1)
    wb = jnp.concatenate([wcol, bcol], axis=1).astype(jnp.float32)

    normed_half = pl.pallas_call(
        _norm_body,
        grid=(G1,),
        in_specs=[
            pl.BlockSpec((EMB, C2), lambda i: (0, i)),
            pl.BlockSpec((128, 2), lambda i: (0, 0)),
            pl.BlockSpec((128, 2), lambda i: (0, 0)),
        ],
        out_specs=pl.BlockSpec((EMB, C2), lambda i: (0, i)),
        out_shape=jax.ShapeDtypeStruct((EMB, N), jnp.float32),
        compiler_params=pltpu.CompilerParams(
            dimension_semantics=("arbitrary",),
        ),
    )(out_a, sums, wb)

    normed_t = pl.pallas_call(
        _norm_body_b,
        grid=(G1,),
        in_specs=[
            pl.BlockSpec(memory_space=pl.ANY),
            pl.BlockSpec((EMB, C2), lambda i: (0, i)),
            pl.BlockSpec((128, 2), lambda i: (0, 0)),
            pl.BlockSpec((128, 2), lambda i: (0, 0)),
        ],
        out_specs=pl.BlockSpec((EMB, C2), lambda i: (0, G1 + i)),
        out_shape=jax.ShapeDtypeStruct((EMB, N), jnp.float32),
        input_output_aliases={0: 0},
        compiler_params=pltpu.CompilerParams(
            dimension_semantics=("arbitrary",),
        ),
    )(normed_half, out_b, sums, wb)

    return (normed_t.T, pos_t.T)


# speedup vs baseline: 1.0641x; 1.0641x over previous
import functools

import jax
import jax.numpy as jnp
from jax import lax
from jax.experimental import pallas as pl
from jax.experimental.pallas import tpu as pltpu
from jax.experimental.pallas import tpu_sc as plsc

N = 819200
EMB = 64
HALF = EMB // 2
N2 = N // 2

NC = 2
NS = 16
NW = NC * NS
ROWS_PER_W = N // NW
CHUNK = 128
CHUNKS_PER_W = ROWS_PER_W // CHUNK
KBUF = 8
OUTER = CHUNKS_PER_W // KBUF

CP = 32768
GP = N // CP
C2 = 16384
G1 = N2 // C2


def _sc_gather(idx3, table):
    mesh = plsc.VectorSubcoreMesh(core_axis_name="c", subcore_axis_name="s")

    @functools.partial(
        pl.kernel,
        out_type=jax.ShapeDtypeStruct((N2, 128), jnp.float32),
        mesh=mesh,
        compiler_params=pltpu.CompilerParams(use_tc_tiling_on_sc=False),
        scratch_types=[
            pltpu.VMEM((CHUNKS_PER_W, CHUNK), jnp.int32),
            pltpu.VMEM((KBUF, CHUNK, EMB), jnp.float32),
            pltpu.SemaphoreType.DMA,
            pltpu.SemaphoreType.DMA,
        ],
    )
    def gather_kernel(idx_hbm, table_hbm, out_hbm, idx_v, rows_v, semg, semo):
        wid = lax.axis_index("s") * NC + lax.axis_index("c")
        pltpu.sync_copy(idx_hbm.at[wid], idx_v)
        half = wid // (NW // 2)
        lane0 = half * EMB
        row2_0 = (wid % (NW // 2)) * ROWS_PER_W

        def body(t, carry):
            @pl.when(t > 0)
            def _():
                for i in range(KBUF):
                    pltpu.make_async_copy(
                        rows_v.at[i],
                        out_hbm.at[pl.ds(0, CHUNK), pl.ds(lane0, EMB)],
                        semo,
                    ).wait()

            for i in range(KBUF):
                c = t * KBUF + i
                pltpu.make_async_copy(
                    table_hbm.at[idx_v.at[c]], rows_v.at[i], semg
                ).start()
            for i in range(KBUF):
                pltpu.make_async_copy(
                    table_hbm.at[idx_v.at[0]], rows_v.at[i], semg
                ).wait()
            for i in range(KBUF):
                c = t * KBUF + i
                pltpu.make_async_copy(
                    rows_v.at[i],
                    out_hbm.at[
                        pl.ds(row2_0 + c * CHUNK, CHUNK), pl.ds(lane0, EMB)
                    ],
                    semo,
                ).start()
            return carry

        lax.fori_loop(0, OUTER, body, 0)
        for i in range(KBUF):
            pltpu.make_async_copy(
                rows_v.at[i],
                out_hbm.at[pl.ds(0, CHUNK), pl.ds(lane0, EMB)],
                semo,
            ).wait()

    return gather_kernel(idx3, table)


def _pos_body(idx_ref, fcol_ref, pos_ref):
    x = idx_ref[...].reshape(1, CP).astype(jnp.float32)
    arg = fcol_ref[0:HALF, :] * x
    s = jnp.sin(arg)
    c = jnp.cos(arg)
    pos_ref[...] = jnp.concatenate([s, c], axis=0)


def _sums_body(pos_a_ref, pos_b_ref, emb_ref, out_a_ref, out_b_ref, sum_ref):
    step = pl.program_id(0)
    pos = jnp.concatenate([pos_a_ref[...], pos_b_ref[...]], axis=0)
    out = emb_ref[...].T + pos
    out_a_ref[...] = out[0:64, :]
    out_b_ref[...] = out[64:128, :]
    ps = jnp.sum(out, axis=1, keepdims=True)
    pq = jnp.sum(out * out, axis=1, keepdims=True)

    @pl.when(step == 0)
    def _():
        sum_ref[...] = jnp.zeros((128, 2), jnp.float32)

    sum_ref[:, 0:1] += ps
    sum_ref[:, 1:2] += pq


def _norm_body(out_ref, sum_ref, wb_ref, normed_ref):
    s1 = jnp.sum(sum_ref[:, 0:1])
    s2 = jnp.sum(sum_ref[:, 1:2])
    total = float(N * EMB)
    mean = s1 / total
    var = s2 / total - mean * mean
    denom = jnp.sqrt(var) + 1e-5
    scale = wb_ref[0:64, 0:1] / denom
    shift = wb_ref[0:64, 1:2]
    normed_ref[...] = (out_ref[...] - mean) * scale + shift


def _norm_body_b(carry_ref, out_ref, sum_ref, wb_ref, normed_ref):
    del carry_ref
    _norm_body(out_ref, sum_ref, wb_ref, normed_ref)


def kernel(input_tensor, table, weight, bias):
    freq = (1e-4) ** jnp.linspace(0.0, 1.0, HALF)
    freqf = jnp.concatenate([freq, freq])
    fcol = freqf.reshape(EMB, 1).astype(jnp.float32)

    pos_t = pl.pallas_call(
        _pos_body,
        grid=(GP,),
        in_specs=[
            pl.BlockSpec((CP,), lambda i: (i,)),
            pl.BlockSpec((EMB, 1), lambda i: (0, 0)),
        ],
        out_specs=pl.BlockSpec((EMB, CP), lambda i: (0, i)),
        out_shape=jax.ShapeDtypeStruct((EMB, N), jnp.float32),
        compiler_params=pltpu.CompilerParams(
            dimension_semantics=("arbitrary",),
        ),
    )(input_tensor, fcol)

    idx3 = input_tensor.reshape(NW, CHUNKS_PER_W, CHUNK)
    emb_c = _sc_gather(idx3, table)

    out_a, out_b, sums = pl.pallas_call(
        _sums_body,
        grid=(G1,),
        in_specs=[
            pl.BlockSpec((EMB, C2), lambda i: (0, i)),
            pl.BlockSpec((EMB, C2), lambda i: (0, G1 + i)),
            pl.BlockSpec((C2, 128), lambda i: (i, 0)),
        ],
        out_specs=[
            pl.BlockSpec((EMB, C2), lambda i: (0, i)),
            pl.BlockSpec((EMB, C2), lambda i: (0, i)),
            pl.BlockSpec((128, 2), lambda i: (0, 0)),
        ],
        out_shape=[
            jax.ShapeDtypeStruct((EMB, N2), jnp.float32),
            jax.ShapeDtypeStruct((EMB, N2), jnp.float32),
            jax.ShapeDtypeStruct((128, 2), jnp.float32),
        ],
        compiler_params=pltpu.CompilerParams(
            dimension_semantics=("arbitrary",),
        ),
    )(pos_t, pos_t, emb_c)

    wcol = jnp.concatenate([weight, weight]).reshape(128, 1)
    bcol = jnp.concatenate([bias, bias]).reshape(128, 1)
    wb = jnp.concatenate([wcol, bcol], axis=1).astype(jnp.float32)

    normed_half = pl.pallas_call(
        _norm_body,
        grid=(G1,),
        in_specs=[
            pl.BlockSpec((EMB, C2), lambda i: (0, i)),
            pl.BlockSpec((128, 2), lambda i: (0, 0)),
            pl.BlockSpec((128, 2), lambda i: (0, 0)),
        ],
        out_specs=pl.BlockSpec((EMB, C2), lambda i: (0, i)),
        out_shape=jax.ShapeDtypeStruct((EMB, N), jnp.float32),
        compiler_params=pltpu.CompilerParams(
            dimension_semantics=("arbitrary",),
        ),
    )(out_a, sums, wb)

    normed_t = pl.pallas_call(
        _norm_body_b,
        grid=(G1,),
        in_specs=[
            pl.BlockSpec(memory_space=pl.ANY),
            pl.BlockSpec((EMB, C2), lambda i: (0, i)),
            pl.BlockSpec((128, 2), lambda i: (0, 0)),
            pl.BlockSpec((128, 2), lambda i: (0, 0)),
        ],
        out_specs=pl.BlockSpec((EMB, C2), lambda i: (0, G1 + i)),
        out_shape=jax.ShapeDtypeStruct((EMB, N), jnp.float32),
        input_output_aliases={0: 0},
        compiler_params=pltpu.CompilerParams(
            dimension_semantics=("arbitrary",),
        ),
    )(normed_half, out_b, sums, wb)

    return (normed_t.T, pos_t.T)
